# baseline (speedup 1.0000x reference)
import functools

import jax
import jax.numpy as jnp
from jax import lax
from jax.experimental import pallas as pl
from jax.experimental.pallas import tpu as pltpu
from jax.experimental.pallas import tpu_sc as plsc

N = 10000
D = 128
DH = 64
NTILES = 16
ROWS_PER_TILE = 640
NPAD = NTILES * ROWS_PER_TILE
DUMMY = 10200
E = 320000
CH = 128
WIN = 16
NWIN = 10
NWIN_HBM = 10
NCH = WIN * NWIN
NCH_D = 79
_BN_SCALE = float(1.0 / (1.0 + 1e-5) ** 0.5)

_mesh = plsc.VectorSubcoreMesh(core_axis_name="c", subcore_axis_name="s")


def _sc_degree(didx):
    @functools.partial(
        pl.kernel,
        out_type=jax.ShapeDtypeStruct((2 * NPAD,), jnp.float32),
        mesh=_mesh,
        scratch_types=[
            pltpu.VMEM_SHARED((NPAD,), jnp.float32),
            pltpu.VMEM((NCH_D, CH), jnp.int32),
            pltpu.VMEM((CH,), jnp.float32),
            pltpu.VMEM((ROWS_PER_TILE,), jnp.float32),
        ],
    )
    def k(di_hbm, out_hbm, dacc, didx_v, ones_v, zbuf):
        c = lax.axis_index("c")
        s = lax.axis_index("s")
        r0 = s * ROWS_PER_TILE

        @pl.loop(0, ROWS_PER_TILE, step=16)
        def _(i):
            zbuf[pl.ds(i, 16)] = jnp.zeros((16,), jnp.float32)

        pltpu.sync_copy(zbuf, dacc.at[pl.ds(r0, ROWS_PER_TILE)])
        pltpu.sync_copy(di_hbm.at[c, s], didx_v)

        @pl.loop(0, CH, step=16)
        def _(i):
            ones_v[pl.ds(i, 16)] = jnp.full((16,), 1.0, jnp.float32)

        plsc.subcore_barrier()

        @pl.loop(0, NCH_D)
        def _(j):
            pltpu.sync_copy(ones_v, dacc.at[didx_v.at[j]], add=True)

        plsc.subcore_barrier()
        pltpu.sync_copy(dacc.at[pl.ds(r0, ROWS_PER_TILE)], zbuf)
        pltpu.sync_copy(zbuf, out_hbm.at[pl.ds(c * NPAD + r0, ROWS_PER_TILE)])

    return k(didx)


def _sc_scatter(p_half, src_t, dst_t, zeros_h):
    @functools.partial(
        pl.kernel,
        out_type=jax.ShapeDtypeStruct((2, NPAD, DH), jnp.float32),
        mesh=_mesh,
        compiler_params=pltpu.CompilerParams(use_tc_tiling_on_sc=False),
        scratch_types=[
            pltpu.VMEM_SHARED((NPAD, DH), jnp.float32),
            pltpu.VMEM_SHARED((NPAD, DH), jnp.float32),
            pltpu.VMEM((WIN, CH), jnp.int32),
            pltpu.VMEM((WIN, CH), jnp.int32),
            pltpu.VMEM((CH, DH), jnp.float32),
            pltpu.VMEM((CH, DH), jnp.float32),
            pltpu.VMEM((CH, DH), jnp.float32),
            pltpu.VMEM((CH, DH), jnp.float32),
            pltpu.SemaphoreType.DMA,
            pltpu.SemaphoreType.DMA,
            pltpu.SemaphoreType.DMA,
            pltpu.SemaphoreType.DMA,
        ],
    )
    def k(p_hbm, s_hbm, d_hbm, z_hbm, out_hbm, tbl, acc, sidx, didx,
          buf0, buf1, buf2, buf3, sem0, sem1, sem2, sem3):
        c = lax.axis_index("c")
        s = lax.axis_index("s")
        r0 = s * ROWS_PER_TILE
        pltpu.sync_copy(p_hbm.at[c, pl.ds(r0, ROWS_PER_TILE)],
                        tbl.at[pl.ds(r0, ROWS_PER_TILE)])
        pltpu.sync_copy(z_hbm.at[pl.ds(r0, ROWS_PER_TILE)],
                        acc.at[pl.ds(r0, ROWS_PER_TILE)])
        plsc.subcore_barrier()
        ptab = p_hbm.at[c]

        bufs = (buf0, buf1, buf2, buf3)
        sems = (sem0, sem1, sem2, sem3)

        def window(w, gsrc):
            pltpu.sync_copy(s_hbm.at[s, pl.ds(w * WIN, WIN)], sidx)
            pltpu.sync_copy(d_hbm.at[s, pl.ds(w * WIN, WIN)], didx)
            for b in range(4):
                pltpu.async_copy(gsrc.at[sidx.at[b]], bufs[b], sems[b])

            @pl.loop(0, WIN - 4, step=4)
            def _(j):
                for b in range(4):
                    pltpu.make_async_copy(gsrc.at[sidx.at[j + b]], bufs[b],
                                          sems[b]).wait()
                    pltpu.sync_copy(bufs[b], acc.at[didx.at[j + b]],
                                    add=True)
                    pltpu.async_copy(gsrc.at[sidx.at[j + b + 4]], bufs[b],
                                     sems[b])

            for b in range(4):
                pltpu.make_async_copy(gsrc.at[sidx.at[WIN - 4 + b]],
                                      bufs[b], sems[b]).wait()
                pltpu.sync_copy(bufs[b], acc.at[didx.at[WIN - 4 + b]],
                                add=True)

        @pl.loop(0, NWIN_HBM)
        def _(w):
            window(w, ptab)

        @pl.loop(NWIN_HBM, NWIN)
        def _(w):
            window(w, tbl)

        plsc.subcore_barrier()
        pltpu.sync_copy(acc.at[pl.ds(r0, ROWS_PER_TILE)],
                        out_hbm.at[c, pl.ds(r0, ROWS_PER_TILE)])

    return k(p_half, src_t, dst_t, zeros_h)


NBLK = 8
BR = NPAD // NBLK

_spec_h = pl.BlockSpec((2, BR, DH), lambda i: (0, i, 0))
_spec_r = pl.BlockSpec((BR, D), lambda i: (i, 0))
_spec_d = pl.BlockSpec((BR, 1), lambda i: (i, 0))
_spec_dg = pl.BlockSpec((2, BR, 1), lambda i: (0, i, 0))
_spec_w = pl.BlockSpec((D, D), lambda i: (0, 0))
_spec_v = pl.BlockSpec((1, D), lambda i: (0, 0))


def _tc_first(x_pad, W1, degp):
    def body(x_ref, w_ref, dg_ref, p_ref, dinv_ref):
        deg = dg_ref[0] + dg_ref[1] + 1.0
        dinv = lax.rsqrt(deg)
        h = jnp.dot(x_ref[...], w_ref[...],
                    preferred_element_type=jnp.float32,
                    precision=lax.Precision.HIGHEST)
        p = h * dinv
        p_ref[0] = p[:, :DH]
        p_ref[1] = p[:, DH:]
        dinv_ref[...] = dinv

    return pl.pallas_call(
        body,
        grid=(NBLK,),
        in_specs=[_spec_r, _spec_w, _spec_dg],
        out_specs=(_spec_h, _spec_d),
        out_shape=(jax.ShapeDtypeStruct((2, NPAD, DH), jnp.float32),
                   jax.ShapeDtypeStruct((NPAD, 1), jnp.float32)),
    )(x_pad, W1, degp)


def _tc_mid(s_in, p_in, dinv, W, b, g, be):
    def body(s_ref, p_ref, di_ref, w_ref, b_ref, g_ref, be_ref, o_ref):
        dinv = di_ref[...]
        t = jnp.concatenate([s_ref[0] + p_ref[0], s_ref[1] + p_ref[1]],
                            axis=1)
        z = t * dinv + b_ref[...]
        z = z * (g_ref[...] * _BN_SCALE) + be_ref[...]
        z = jnp.maximum(z, 0.0)
        h = jnp.dot(z, w_ref[...],
                    preferred_element_type=jnp.float32,
                    precision=lax.Precision.HIGHEST)
        p = h * dinv
        o_ref[0] = p[:, :DH]
        o_ref[1] = p[:, DH:]

    return pl.pallas_call(
        body,
        grid=(NBLK,),
        in_specs=[_spec_h, _spec_h, _spec_d, _spec_w, _spec_v, _spec_v,
                  _spec_v],
        out_specs=_spec_h,
        out_shape=jax.ShapeDtypeStruct((2, NPAD, DH), jnp.float32),
    )(s_in, p_in, dinv, W, b.reshape(1, D), g.reshape(1, D),
      be.reshape(1, D))


def _tc_final(s_in, p_in, dinv, b):
    def body(s_ref, p_ref, di_ref, b_ref, o_ref):
        z = jnp.concatenate([s_ref[0] + p_ref[0], s_ref[1] + p_ref[1]],
                            axis=1)
        z = z * di_ref[...] + b_ref[...]
        m = jnp.max(z, axis=1, keepdims=True)
        zs = z - m
        lse = jnp.log(jnp.sum(jnp.exp(zs), axis=1, keepdims=True))
        o_ref[...] = zs - lse

    return pl.pallas_call(
        body,
        grid=(NBLK,),
        in_specs=[_spec_h, _spec_h, _spec_d, _spec_v],
        out_specs=_spec_r,
        out_shape=jax.ShapeDtypeStruct((NPAD, D), jnp.float32),
    )(s_in, p_in, dinv, b.reshape(1, D))


def kernel(x, edge_index, W1, b1, g1, be1, W2, b2, g2, be2, W3, b3):
    src = edge_index[0]
    dst = edge_index[1]

    ept = E // NTILES
    pad_t = NCH * CH - ept
    src_t = jnp.pad(src.reshape(NTILES, ept), ((0, 0), (0, pad_t)),
                    constant_values=DUMMY).reshape(NTILES, NCH, CH)
    dst_t = jnp.pad(dst.reshape(NTILES, ept), ((0, 0), (0, pad_t)),
                    constant_values=DUMMY).reshape(NTILES, NCH, CH)

    pad_d = NCH_D * CH * NTILES - E // 2
    padd = jnp.full((2, pad_d), DUMMY, jnp.int32)
    didx = jnp.concatenate([dst.reshape(2, E // 2), padd], axis=1)
    didx = didx.reshape(2, NTILES, NCH_D, CH)

    zeros_h = jnp.zeros((NPAD, DH), jnp.float32)
    x_pad = jnp.pad(x, ((0, NPAD - N), (0, 0)))

    degp = _sc_degree(didx).reshape(2, NPAD, 1)
    p1, dinv = _tc_first(x_pad, W1, degp)
    s1 = _sc_scatter(p1, src_t, dst_t, zeros_h)
    p2 = _tc_mid(s1, p1, dinv, W2, b1, g1, be1)
    s2 = _sc_scatter(p2, src_t, dst_t, zeros_h)
    p3 = _tc_mid(s2, p2, dinv, W3, b2, g2, be2)
    s3 = _sc_scatter(p3, src_t, dst_t, zeros_h)
    out = _tc_final(s3, p3, dinv, b3)
    return out[:N]


# speedup vs baseline: 1.6878x; 1.6878x over previous
import functools

import jax
import jax.numpy as jnp
from jax import lax
from jax.experimental import pallas as pl
from jax.experimental.pallas import tpu as pltpu
from jax.experimental.pallas import tpu_sc as plsc

N = 10000
D = 128
DH = 64
NTILES = 16
ROWS_PER_TILE = 640
NPAD = NTILES * ROWS_PER_TILE
DUMMY = 10200
E = 320000
CH = 128
WIN = 16
NWIN = 10
NWIN_HBM = 9
NCH = WIN * NWIN
NCH_D = 79
_BN_SCALE = float(1.0 / (1.0 + 1e-5) ** 0.5)

_mesh = plsc.VectorSubcoreMesh(core_axis_name="c", subcore_axis_name="s")


def _sc_degree(didx):
    @functools.partial(
        pl.kernel,
        out_type=jax.ShapeDtypeStruct((2 * NPAD,), jnp.float32),
        mesh=_mesh,
        scratch_types=[
            pltpu.VMEM_SHARED((NPAD,), jnp.float32),
            pltpu.VMEM((NCH_D, CH), jnp.int32),
            pltpu.VMEM((CH,), jnp.float32),
            pltpu.VMEM((ROWS_PER_TILE,), jnp.float32),
        ],
    )
    def k(di_hbm, out_hbm, dacc, didx_v, ones_v, zbuf):
        c = lax.axis_index("c")
        s = lax.axis_index("s")
        r0 = s * ROWS_PER_TILE

        @pl.loop(0, ROWS_PER_TILE, step=16)
        def _(i):
            zbuf[pl.ds(i, 16)] = jnp.zeros((16,), jnp.float32)

        pltpu.sync_copy(zbuf, dacc.at[pl.ds(r0, ROWS_PER_TILE)])
        pltpu.sync_copy(di_hbm.at[c, s], didx_v)

        @pl.loop(0, CH, step=16)
        def _(i):
            ones_v[pl.ds(i, 16)] = jnp.full((16,), 1.0, jnp.float32)

        plsc.subcore_barrier()

        @pl.loop(0, NCH_D)
        def _(j):
            pltpu.sync_copy(ones_v, dacc.at[didx_v.at[j]], add=True)

        plsc.subcore_barrier()
        pltpu.sync_copy(dacc.at[pl.ds(r0, ROWS_PER_TILE)], zbuf)
        pltpu.sync_copy(zbuf, out_hbm.at[pl.ds(c * NPAD + r0, ROWS_PER_TILE)])

    return k(didx)


def _sc_scatter(p_half, src_t, dst_t, zeros_h):
    @functools.partial(
        pl.kernel,
        out_type=jax.ShapeDtypeStruct((2, NPAD, DH), jnp.float32),
        mesh=_mesh,
        compiler_params=pltpu.CompilerParams(use_tc_tiling_on_sc=False),
        scratch_types=[
            pltpu.VMEM_SHARED((NPAD, DH), jnp.float32),
            pltpu.VMEM_SHARED((NPAD, DH), jnp.float32),
            pltpu.VMEM((WIN, CH), jnp.int32),
            pltpu.VMEM((WIN, CH), jnp.int32),
            pltpu.VMEM((CH, DH), jnp.float32),
            pltpu.VMEM((CH, DH), jnp.float32),
            pltpu.VMEM((CH, DH), jnp.float32),
            pltpu.VMEM((CH, DH), jnp.float32),
            pltpu.SemaphoreType.DMA,
            pltpu.SemaphoreType.DMA,
            pltpu.SemaphoreType.DMA,
            pltpu.SemaphoreType.DMA,
        ],
    )
    def k(p_hbm, s_hbm, d_hbm, z_hbm, out_hbm, tbl, acc, sidx, didx,
          buf0, buf1, buf2, buf3, sem0, sem1, sem2, sem3):
        c = lax.axis_index("c")
        s = lax.axis_index("s")
        r0 = s * ROWS_PER_TILE
        pltpu.sync_copy(p_hbm.at[c, pl.ds(r0, ROWS_PER_TILE)],
                        tbl.at[pl.ds(r0, ROWS_PER_TILE)])
        pltpu.sync_copy(z_hbm.at[pl.ds(r0, ROWS_PER_TILE)],
                        acc.at[pl.ds(r0, ROWS_PER_TILE)])
        plsc.subcore_barrier()
        ptab = p_hbm.at[c]

        bufs = (buf0, buf1, buf2, buf3)
        sems = (sem0, sem1, sem2, sem3)

        def window(w, gsrc):
            pltpu.sync_copy(s_hbm.at[s, pl.ds(w * WIN, WIN)], sidx)
            pltpu.sync_copy(d_hbm.at[s, pl.ds(w * WIN, WIN)], didx)
            for b in range(4):
                pltpu.async_copy(gsrc.at[sidx.at[b]], bufs[b], sems[b])

            @pl.loop(0, WIN - 4, step=4)
            def _(j):
                for b in range(4):
                    pltpu.make_async_copy(gsrc.at[sidx.at[j + b]], bufs[b],
                                          sems[b]).wait()
                    pltpu.sync_copy(bufs[b], acc.at[didx.at[j + b]],
                                    add=True)
                    pltpu.async_copy(gsrc.at[sidx.at[j + b + 4]], bufs[b],
                                     sems[b])

            for b in range(4):
                pltpu.make_async_copy(gsrc.at[sidx.at[WIN - 4 + b]],
                                      bufs[b], sems[b]).wait()
                pltpu.sync_copy(bufs[b], acc.at[didx.at[WIN - 4 + b]],
                                add=True)

        @pl.loop(0, NWIN_HBM)
        def _(w):
            window(w, ptab)

        @pl.loop(NWIN_HBM, NWIN)
        def _(w):
            window(w, tbl)

        plsc.subcore_barrier()
        pltpu.sync_copy(acc.at[pl.ds(r0, ROWS_PER_TILE)],
                        out_hbm.at[c, pl.ds(r0, ROWS_PER_TILE)])

    return k(p_half, src_t, dst_t, zeros_h)


NBLK = 8
BR = NPAD // NBLK

_spec_h = pl.BlockSpec((2, BR, DH), lambda i: (0, i, 0))
_spec_r = pl.BlockSpec((BR, D), lambda i: (i, 0))
_spec_d = pl.BlockSpec((BR, 1), lambda i: (i, 0))
_spec_dg = pl.BlockSpec((2, BR, 1), lambda i: (0, i, 0))
_spec_w = pl.BlockSpec((D, D), lambda i: (0, 0))
_spec_v = pl.BlockSpec((1, D), lambda i: (0, 0))


def _tc_first(x_pad, W1, degp):
    def body(x_ref, w_ref, dg_ref, p_ref, dinv_ref):
        deg = dg_ref[0] + dg_ref[1] + 1.0
        dinv = lax.rsqrt(deg)
        h = jnp.dot(x_ref[...], w_ref[...],
                    preferred_element_type=jnp.float32,
                    precision=lax.Precision.HIGHEST)
        p = h * dinv
        p_ref[0] = p[:, :DH]
        p_ref[1] = p[:, DH:]
        dinv_ref[...] = dinv

    return pl.pallas_call(
        body,
        grid=(NBLK,),
        in_specs=[_spec_r, _spec_w, _spec_dg],
        out_specs=(_spec_h, _spec_d),
        out_shape=(jax.ShapeDtypeStruct((2, NPAD, DH), jnp.float32),
                   jax.ShapeDtypeStruct((NPAD, 1), jnp.float32)),
    )(x_pad, W1, degp)


def _tc_mid(s_in, p_in, dinv, W, b, g, be):
    def body(s_ref, p_ref, di_ref, w_ref, b_ref, g_ref, be_ref, o_ref):
        dinv = di_ref[...]
        t = jnp.concatenate([s_ref[0] + p_ref[0], s_ref[1] + p_ref[1]],
                            axis=1)
        z = t * dinv + b_ref[...]
        z = z * (g_ref[...] * _BN_SCALE) + be_ref[...]
        z = jnp.maximum(z, 0.0)
        h = jnp.dot(z, w_ref[...],
                    preferred_element_type=jnp.float32,
                    precision=lax.Precision.HIGHEST)
        p = h * dinv
        o_ref[0] = p[:, :DH]
        o_ref[1] = p[:, DH:]

    return pl.pallas_call(
        body,
        grid=(NBLK,),
        in_specs=[_spec_h, _spec_h, _spec_d, _spec_w, _spec_v, _spec_v,
                  _spec_v],
        out_specs=_spec_h,
        out_shape=jax.ShapeDtypeStruct((2, NPAD, DH), jnp.float32),
    )(s_in, p_in, dinv, W, b.reshape(1, D), g.reshape(1, D),
      be.reshape(1, D))


def _tc_final(s_in, p_in, dinv, b):
    def body(s_ref, p_ref, di_ref, b_ref, o_ref):
        z = jnp.concatenate([s_ref[0] + p_ref[0], s_ref[1] + p_ref[1]],
                            axis=1)
        z = z * di_ref[...] + b_ref[...]
        m = jnp.max(z, axis=1, keepdims=True)
        zs = z - m
        lse = jnp.log(jnp.sum(jnp.exp(zs), axis=1, keepdims=True))
        o_ref[...] = zs - lse

    return pl.pallas_call(
        body,
        grid=(NBLK,),
        in_specs=[_spec_h, _spec_h, _spec_d, _spec_v],
        out_specs=_spec_r,
        out_shape=jax.ShapeDtypeStruct((NPAD, D), jnp.float32),
    )(s_in, p_in, dinv, b.reshape(1, D))


def kernel(x, edge_index, W1, b1, g1, be1, W2, b2, g2, be2, W3, b3):
    src = edge_index[0]
    dst = edge_index[1]

    ept = E // NTILES
    pad_t = NCH * CH - ept
    src_t = jnp.pad(src.reshape(NTILES, ept), ((0, 0), (0, pad_t)),
                    constant_values=DUMMY).reshape(NTILES, NCH, CH)
    dst_t = jnp.pad(dst.reshape(NTILES, ept), ((0, 0), (0, pad_t)),
                    constant_values=DUMMY).reshape(NTILES, NCH, CH)

    pad_d = NCH_D * CH * NTILES - E // 2
    padd = jnp.full((2, pad_d), DUMMY, jnp.int32)
    didx = jnp.concatenate([dst.reshape(2, E // 2), padd], axis=1)
    didx = didx.reshape(2, NTILES, NCH_D, CH)

    zeros_h = jnp.zeros((NPAD, DH), jnp.float32)
    x_pad = jnp.pad(x, ((0, NPAD - N), (0, 0)))

    degp = _sc_degree(didx).reshape(2, NPAD, 1)
    p1, dinv = _tc_first(x_pad, W1, degp)
    s1 = _sc_scatter(p1, src_t, dst_t, zeros_h)
    p2 = _tc_mid(s1, p1, dinv, W2, b1, g1, be1)
    s2 = _sc_scatter(p2, src_t, dst_t, zeros_h)
    p3 = _tc_mid(s2, p2, dinv, W3, b2, g2, be2)
    s3 = _sc_scatter(p3, src_t, dst_t, zeros_h)
    out = _tc_final(s3, p3, dinv, b3)
    return out[:N]


# speedup vs baseline: 1.6989x; 1.0066x over previous
import functools

import jax
import jax.numpy as jnp
from jax import lax
from jax.experimental import pallas as pl
from jax.experimental.pallas import tpu as pltpu
from jax.experimental.pallas import tpu_sc as plsc

N = 10000
D = 128
DH = 64
NTILES = 16
ROWS_PER_TILE = 640
NPAD = NTILES * ROWS_PER_TILE
DUMMY = 10200
E = 320000
CH = 128
WIN = 16
NWIN = 10
NWIN_HBM = 9
NCH = WIN * NWIN
NCH_D = 79
_BN_SCALE = float(1.0 / (1.0 + 1e-5) ** 0.5)

_mesh = plsc.VectorSubcoreMesh(core_axis_name="c", subcore_axis_name="s")


def _sc_degree(didx):
    @functools.partial(
        pl.kernel,
        out_type=jax.ShapeDtypeStruct((2 * NPAD,), jnp.float32),
        mesh=_mesh,
        scratch_types=[
            pltpu.VMEM_SHARED((NPAD,), jnp.float32),
            pltpu.VMEM((NCH_D, CH), jnp.int32),
            pltpu.VMEM((CH,), jnp.float32),
            pltpu.VMEM((ROWS_PER_TILE,), jnp.float32),
        ],
    )
    def k(di_hbm, out_hbm, dacc, didx_v, ones_v, zbuf):
        c = lax.axis_index("c")
        s = lax.axis_index("s")
        r0 = s * ROWS_PER_TILE

        @pl.loop(0, ROWS_PER_TILE, step=16)
        def _(i):
            zbuf[pl.ds(i, 16)] = jnp.zeros((16,), jnp.float32)

        pltpu.sync_copy(zbuf, dacc.at[pl.ds(r0, ROWS_PER_TILE)])
        pltpu.sync_copy(di_hbm.at[c, s], didx_v)

        @pl.loop(0, CH, step=16)
        def _(i):
            ones_v[pl.ds(i, 16)] = jnp.full((16,), 1.0, jnp.float32)

        plsc.subcore_barrier()

        @pl.loop(0, NCH_D)
        def _(j):
            pltpu.sync_copy(ones_v, dacc.at[didx_v.at[j]], add=True)

        plsc.subcore_barrier()
        pltpu.sync_copy(dacc.at[pl.ds(r0, ROWS_PER_TILE)], zbuf)
        pltpu.sync_copy(zbuf, out_hbm.at[pl.ds(c * NPAD + r0, ROWS_PER_TILE)])

    return k(didx)


def _sc_scatter(p_half, src_t, dst_t, zeros_h):
    @functools.partial(
        pl.kernel,
        out_type=jax.ShapeDtypeStruct((2, NPAD, DH), jnp.float32),
        mesh=_mesh,
        compiler_params=pltpu.CompilerParams(use_tc_tiling_on_sc=False),
        scratch_types=[
            pltpu.VMEM_SHARED((NPAD, DH), jnp.float32),
            pltpu.VMEM_SHARED((NPAD, DH), jnp.float32),
            pltpu.VMEM((WIN, CH), jnp.int32),
            pltpu.VMEM((WIN, CH), jnp.int32),
            pltpu.VMEM((CH, DH), jnp.float32),
            pltpu.VMEM((CH, DH), jnp.float32),
            pltpu.VMEM((CH, DH), jnp.float32),
            pltpu.VMEM((CH, DH), jnp.float32),
            pltpu.SemaphoreType.DMA,
            pltpu.SemaphoreType.DMA,
            pltpu.SemaphoreType.DMA,
            pltpu.SemaphoreType.DMA,
        ],
    )
    def k(p_hbm, s_hbm, d_hbm, z_hbm, out_hbm, tbl, acc, sidx, didx,
          buf0, buf1, buf2, buf3, sem0, sem1, sem2, sem3):
        c = lax.axis_index("c")
        s = lax.axis_index("s")
        r0 = s * ROWS_PER_TILE
        pltpu.sync_copy(p_hbm.at[c, pl.ds(r0, ROWS_PER_TILE)],
                        tbl.at[pl.ds(r0, ROWS_PER_TILE)])
        pltpu.sync_copy(z_hbm.at[pl.ds(r0, ROWS_PER_TILE)],
                        acc.at[pl.ds(r0, ROWS_PER_TILE)])
        plsc.subcore_barrier()
        ptab = p_hbm.at[c]

        bufs = (buf0, buf1, buf2, buf3)
        sems = (sem0, sem1, sem2, sem3)

        def window(w, gsrc):
            pltpu.sync_copy(s_hbm.at[s, pl.ds(w * WIN, WIN)], sidx)
            pltpu.sync_copy(d_hbm.at[s, pl.ds(w * WIN, WIN)], didx)
            for b in range(4):
                pltpu.async_copy(gsrc.at[sidx.at[b]], bufs[b], sems[b])

            @pl.loop(0, WIN - 4, step=4)
            def _(j):
                for b in range(4):
                    pltpu.make_async_copy(gsrc.at[sidx.at[j + b]], bufs[b],
                                          sems[b]).wait()
                    pltpu.sync_copy(bufs[b], acc.at[didx.at[j + b]],
                                    add=True)
                    pltpu.async_copy(gsrc.at[sidx.at[j + b + 4]], bufs[b],
                                     sems[b])

            for b in range(4):
                pltpu.make_async_copy(gsrc.at[sidx.at[WIN - 4 + b]],
                                      bufs[b], sems[b]).wait()
                pltpu.sync_copy(bufs[b], acc.at[didx.at[WIN - 4 + b]],
                                add=True)

        @pl.loop(0, NWIN_HBM)
        def _(w):
            window(w, ptab)

        @pl.loop(NWIN_HBM, NWIN)
        def _(w):
            window(w, tbl)

        plsc.subcore_barrier()
        pltpu.sync_copy(acc.at[pl.ds(r0, ROWS_PER_TILE)],
                        out_hbm.at[c, pl.ds(r0, ROWS_PER_TILE)])

    return k(p_half, src_t, dst_t, zeros_h)


NBLK = 8
BR = NPAD // NBLK

_spec_h = pl.BlockSpec((2, BR, DH), lambda i: (0, i, 0))
_spec_r = pl.BlockSpec((BR, D), lambda i: (i, 0))
_spec_d = pl.BlockSpec((BR, 1), lambda i: (i, 0))
_spec_dg = pl.BlockSpec((2, BR, 1), lambda i: (0, i, 0))
_spec_w = pl.BlockSpec((D, D), lambda i: (0, 0))
_spec_v = pl.BlockSpec((1, D), lambda i: (0, 0))


def _tc_first(x_pad, W1, degp):
    def body(x_ref, w_ref, dg_ref, p_ref, dinv_ref):
        deg = dg_ref[0] + dg_ref[1] + 1.0
        dinv = lax.rsqrt(deg)
        h = jnp.dot(x_ref[...], w_ref[...],
                    preferred_element_type=jnp.float32,
                    precision=lax.Precision.DEFAULT)
        p = h * dinv
        p_ref[0] = p[:, :DH]
        p_ref[1] = p[:, DH:]
        dinv_ref[...] = dinv

    return pl.pallas_call(
        body,
        grid=(NBLK,),
        in_specs=[_spec_r, _spec_w, _spec_dg],
        out_specs=(_spec_h, _spec_d),
        out_shape=(jax.ShapeDtypeStruct((2, NPAD, DH), jnp.float32),
                   jax.ShapeDtypeStruct((NPAD, 1), jnp.float32)),
    )(x_pad, W1, degp)


def _tc_mid(s_in, p_in, dinv, W, b, g, be):
    def body(s_ref, p_ref, di_ref, w_ref, b_ref, g_ref, be_ref, o_ref):
        dinv = di_ref[...]
        t = jnp.concatenate([s_ref[0] + p_ref[0], s_ref[1] + p_ref[1]],
                            axis=1)
        z = t * dinv + b_ref[...]
        z = z * (g_ref[...] * _BN_SCALE) + be_ref[...]
        z = jnp.maximum(z, 0.0)
        h = jnp.dot(z, w_ref[...],
                    preferred_element_type=jnp.float32,
                    precision=lax.Precision.DEFAULT)
        p = h * dinv
        o_ref[0] = p[:, :DH]
        o_ref[1] = p[:, DH:]

    return pl.pallas_call(
        body,
        grid=(NBLK,),
        in_specs=[_spec_h, _spec_h, _spec_d, _spec_w, _spec_v, _spec_v,
                  _spec_v],
        out_specs=_spec_h,
        out_shape=jax.ShapeDtypeStruct((2, NPAD, DH), jnp.float32),
    )(s_in, p_in, dinv, W, b.reshape(1, D), g.reshape(1, D),
      be.reshape(1, D))


def _tc_final(s_in, p_in, dinv, b):
    def body(s_ref, p_ref, di_ref, b_ref, o_ref):
        z = jnp.concatenate([s_ref[0] + p_ref[0], s_ref[1] + p_ref[1]],
                            axis=1)
        z = z * di_ref[...] + b_ref[...]
        m = jnp.max(z, axis=1, keepdims=True)
        zs = z - m
        lse = jnp.log(jnp.sum(jnp.exp(zs), axis=1, keepdims=True))
        o_ref[...] = zs - lse

    return pl.pallas_call(
        body,
        grid=(NBLK,),
        in_specs=[_spec_h, _spec_h, _spec_d, _spec_v],
        out_specs=_spec_r,
        out_shape=jax.ShapeDtypeStruct((NPAD, D), jnp.float32),
    )(s_in, p_in, dinv, b.reshape(1, D))


def kernel(x, edge_index, W1, b1, g1, be1, W2, b2, g2, be2, W3, b3):
    src = edge_index[0]
    dst = edge_index[1]

    ept = E // NTILES
    pad_t = NCH * CH - ept
    src_t = jnp.pad(src.reshape(NTILES, ept), ((0, 0), (0, pad_t)),
                    constant_values=DUMMY).reshape(NTILES, NCH, CH)
    dst_t = jnp.pad(dst.reshape(NTILES, ept), ((0, 0), (0, pad_t)),
                    constant_values=DUMMY).reshape(NTILES, NCH, CH)

    pad_d = NCH_D * CH * NTILES - E // 2
    padd = jnp.full((2, pad_d), DUMMY, jnp.int32)
    didx = jnp.concatenate([dst.reshape(2, E // 2), padd], axis=1)
    didx = didx.reshape(2, NTILES, NCH_D, CH)

    zeros_h = jnp.zeros((NPAD, DH), jnp.float32)
    x_pad = jnp.pad(x, ((0, NPAD - N), (0, 0)))

    degp = _sc_degree(didx).reshape(2, NPAD, 1)
    p1, dinv = _tc_first(x_pad, W1, degp)
    s1 = _sc_scatter(p1, src_t, dst_t, zeros_h)
    p2 = _tc_mid(s1, p1, dinv, W2, b1, g1, be1)
    s2 = _sc_scatter(p2, src_t, dst_t, zeros_h)
    p3 = _tc_mid(s2, p2, dinv, W3, b2, g2, be2)
    s3 = _sc_scatter(p3, src_t, dst_t, zeros_h)
    out = _tc_final(s3, p3, dinv, b3)
    return out[:N]
